# baseline (speedup 1.0000x reference)
import functools

import jax
import jax.numpy as jnp
from jax import lax
from jax.experimental import pallas as pl
from jax.experimental.pallas import tpu as pltpu
from jax.experimental.pallas import tpu_sc as plsc

VOCAB_SIZE = 100000
EMB_DIM = 128
CONTEXT_SIZE = 8192
BATCH = 4
SEQ_LEN = 8192

NUM_WORKERS = 32
POS_PER_W = SEQ_LEN // NUM_WORKERS
CHUNK = 128
HALVES = POS_PER_W // CHUNK
NCHUNK = BATCH * HALVES
NG = 3
LANES = 16
VREGS_PER_ROW = EMB_DIM // LANES

_mesh = plsc.VectorSubcoreMesh(core_axis_name="c", subcore_axis_name="s")


@functools.partial(
    pl.kernel,
    mesh=_mesh,
    out_type=jax.ShapeDtypeStruct((BATCH, SEQ_LEN, EMB_DIM), jnp.float32),
    scratch_types=[
        pltpu.VMEM((BATCH, HALVES, CHUNK), jnp.int32),
        pltpu.VMEM((POS_PER_W, EMB_DIM), jnp.float32),
        pltpu.VMEM((CHUNK, EMB_DIM), jnp.float32),
        pltpu.VMEM((CHUNK, EMB_DIM), jnp.float32),
        pltpu.VMEM((CHUNK, EMB_DIM), jnp.float32),
        pltpu.VMEM((CHUNK, EMB_DIM), jnp.float32),
        pltpu.VMEM((CHUNK, EMB_DIM), jnp.float32),
        pltpu.SemaphoreType.DMA,
        pltpu.SemaphoreType.DMA,
        pltpu.SemaphoreType.DMA,
        pltpu.SemaphoreType.DMA,
        pltpu.SemaphoreType.DMA,
    ],
)
def _emb_lookup(ids_hbm, tok_hbm, pos_hbm, out_hbm, ids_v, pos_v,
                gbuf0, gbuf1, gbuf2, obuf0, obuf1, g0, g1, g2, o0, o1):
    gbuf = (gbuf0, gbuf1, gbuf2)
    obuf = (obuf0, obuf1)
    gsem = (g0, g1, g2)
    osem = (o0, o1)

    wid = lax.axis_index("s") * 2 + lax.axis_index("c")
    pos0 = wid * POS_PER_W

    for h in range(HALVES):
        pltpu.sync_copy(ids_hbm.at[:, pl.ds(pos0 + h * CHUNK, CHUNK)],
                        ids_v.at[:, h])

    def start_gather(k, buf, sem):
        return [pltpu.async_copy(
            tok_hbm.at[ids_v.at[k // HALVES, k % HALVES, pl.ds(q * 32, 32)]],
            buf.at[pl.ds(q * 32, 32)], sem) for q in range(4)]

    gh = [start_gather(k, gbuf[k], gsem[k]) for k in range(NG)]
    pltpu.sync_copy(pos_hbm.at[pl.ds(pos0, POS_PER_W)], pos_v)

    oh = [None, None]
    for c in range(NCHUNK):
        gb = gbuf[c % NG]
        ob = obuf[c % 2]
        bi, h = divmod(c, HALVES)
        for hnd in gh[c % NG]:
            hnd.wait()
        if c >= 2:
            oh[c % 2].wait()

        hbase = h * CHUNK

        def add_body(i, carry):
            for j in range(VREGS_PER_ROW):
                sl = pl.ds(j * LANES, LANES)
                ob[i, sl] = gb[i, sl] + pos_v[hbase + i, sl]
            return carry

        lax.fori_loop(0, CHUNK, add_body, 0)

        nxt = c + NG
        if nxt < NCHUNK:
            gh[c % NG] = start_gather(nxt, gb, gsem[c % NG])

        oh[c % 2] = pltpu.async_copy(
            ob, out_hbm.at[bi, pl.ds(pos0 + h * CHUNK, CHUNK)], osem[c % 2])

    oh[0].wait()
    oh[1].wait()


def kernel(token_ids, tok_emb, pos_emb):
    return _emb_lookup(token_ids.astype(jnp.int32), tok_emb, pos_emb)


# speedup vs baseline: 1.0300x; 1.0300x over previous
import functools

import jax
import jax.numpy as jnp
from jax import lax
from jax.experimental import pallas as pl
from jax.experimental.pallas import tpu as pltpu
from jax.experimental.pallas import tpu_sc as plsc

VOCAB_SIZE = 100000
EMB_DIM = 128
CONTEXT_SIZE = 8192
BATCH = 4
SEQ_LEN = 8192

NUM_WORKERS = 32
POS_PER_W = SEQ_LEN // NUM_WORKERS
CHUNK = 128
HALVES = POS_PER_W // CHUNK
NCHUNK = BATCH * HALVES
NG = 3
LANES = 16
VREGS_PER_ROW = EMB_DIM // LANES

_mesh = plsc.VectorSubcoreMesh(core_axis_name="c", subcore_axis_name="s")


@functools.partial(
    pl.kernel,
    mesh=_mesh,
    out_type=jax.ShapeDtypeStruct((BATCH, SEQ_LEN, EMB_DIM), jnp.float32),
    scratch_types=[
        pltpu.VMEM((BATCH, POS_PER_W), jnp.int32),
        pltpu.VMEM((POS_PER_W, EMB_DIM), jnp.float32),
        pltpu.VMEM((CHUNK, EMB_DIM), jnp.float32),
        pltpu.VMEM((CHUNK, EMB_DIM), jnp.float32),
        pltpu.VMEM((CHUNK, EMB_DIM), jnp.float32),
        pltpu.VMEM((CHUNK, EMB_DIM), jnp.float32),
        pltpu.VMEM((CHUNK, EMB_DIM), jnp.float32),
        pltpu.SemaphoreType.DMA,
        pltpu.SemaphoreType.DMA,
        pltpu.SemaphoreType.DMA,
        pltpu.SemaphoreType.DMA,
        pltpu.SemaphoreType.DMA,
    ],
)
def _emb_lookup(ids_hbm, tok_hbm, pos_hbm, out_hbm, ids_v, pos_v,
                gbuf0, gbuf1, gbuf2, obuf0, obuf1, g0, g1, g2, o0, o1):
    gbuf = (gbuf0, gbuf1, gbuf2)
    obuf = (obuf0, obuf1)
    gsem = (g0, g1, g2)
    osem = (o0, o1)

    wid = lax.axis_index("s") * 2 + lax.axis_index("c")
    pos0 = wid * POS_PER_W

    pltpu.sync_copy(ids_hbm.at[:, pl.ds(pos0, POS_PER_W)], ids_v)
    ph = pltpu.async_copy(pos_hbm.at[pl.ds(pos0, POS_PER_W)], pos_v, o0)

    def idx_ref(k):
        return ids_v.at[k // HALVES, pl.ds((k % HALVES) * CHUNK, CHUNK)]

    gh = [pltpu.async_copy(tok_hbm.at[idx_ref(k)], gbuf[k], gsem[k])
          for k in range(NG)]
    ph.wait()

    oh = [None, None]
    for c in range(NCHUNK):
        gb = gbuf[c % NG]
        ob = obuf[c % 2]
        bi, h = divmod(c, HALVES)
        gh[c % NG].wait()
        if c >= 2:
            oh[c % 2].wait()

        hbase = h * CHUNK

        def add_body(i, carry):
            for j in range(VREGS_PER_ROW):
                sl = pl.ds(j * LANES, LANES)
                ob[i, sl] = gb[i, sl] + pos_v[hbase + i, sl]
            return carry

        lax.fori_loop(0, CHUNK, add_body, 0)

        nxt = c + NG
        if nxt < NCHUNK:
            gh[c % NG] = pltpu.async_copy(tok_hbm.at[idx_ref(nxt)], gb,
                                          gsem[c % NG])

        oh[c % 2] = pltpu.async_copy(
            ob, out_hbm.at[bi, pl.ds(pos0 + h * CHUNK, CHUNK)], osem[c % 2])

    oh[0].wait()
    oh[1].wait()


def kernel(token_ids, tok_emb, pos_emb):
    return _emb_lookup(token_ids.astype(jnp.int32), tok_emb, pos_emb)
